# baseline (speedup 1.0000x reference)
import jax
import jax.numpy as jnp
from jax.experimental import pallas as pl
from jax.experimental.pallas import tpu as pltpu

DIM = 32
NUM_EMBEDDINGS = 8192
TEMP = 4.0
BLOCK_N = 1024
_EXP2_SCALE = 1.4426950408889634 / TEMP
_AUG = 64


def _fused_body(x_ref, embed_ref, q_ref, ind_ref, acc_ref, e2_ref, eaug_ref,
                iota_ref, maug_ref):
    i = pl.program_id(0)
    xb = x_ref[...]
    emb = embed_ref[...]

    @pl.when(i == 0)
    def _init():
        e2_ref[...] = jnp.sum(emb * emb, axis=0, keepdims=True)
        rows = jax.lax.broadcasted_iota(
            jnp.int32, (_AUG, NUM_EMBEDDINGS), 0)
        eaug_ref[...] = jnp.where(rows == DIM, 1.0, 0.0)
        eaug_ref[0:DIM, :] = emb
        iota_ref[...] = jax.lax.broadcasted_iota(
            jnp.int32, (1, NUM_EMBEDDINGS), 1).astype(jnp.float32)
        mrows = jax.lax.broadcasted_iota(jnp.int32, (8, NUM_EMBEDDINGS), 0)
        maug_ref[...] = jnp.where(mrows == 0, iota_ref[...],
                                  jnp.where(mrows == 1, 1.0, 0.0))
        acc_ref[...] = jnp.zeros((1, 1), jnp.float32)

    x2 = jnp.sum(xb * xb, axis=1, keepdims=True)
    xm2 = jax.lax.dot_general(
        xb * (-2.0), emb, (((1,), (0,)), ((), ())),
        preferred_element_type=jnp.float32)
    dist = (x2 + xm2) + e2_ref[...]

    m = jnp.min(dist, axis=1, keepdims=True)
    mask = jnp.where(dist == m, 1.0, 0.0)
    idxs = jax.lax.dot_general(
        mask, maug_ref[...], (((1,), (1,)), ((), ())),
        preferred_element_type=jnp.float32)
    ind_ref[...] = idxs[:, 0].astype(jnp.int32)
    tie = jnp.any(idxs[:, 1] > 1.5)

    @pl.when(tie)
    def _exact_tiebreak():
        ind_f = jnp.min(
            jnp.where(dist == m, iota_ref[...], float(NUM_EMBEDDINGS)),
            axis=1)
        ind_ref[...] = ind_f.astype(jnp.int32)

    ez = jnp.exp2(dist * (-_EXP2_SCALE))

    qs = jax.lax.dot_general(
        ez, eaug_ref[...], (((1,), (1,)), ((), ())),
        preferred_element_type=jnp.float32)
    q = qs[:, 0:DIM] / qs[:, DIM:DIM + 1]
    q_ref[...] = q

    acc_ref[...] += jnp.sum((q - xb) ** 2).reshape(1, 1)


def kernel(x, embed):
    n = x.shape[0] * x.shape[1]
    flat = x.reshape(n, DIM)
    grid = (n // BLOCK_N,)
    q, ind, acc = pl.pallas_call(
        _fused_body,
        grid=grid,
        in_specs=[
            pl.BlockSpec((BLOCK_N, DIM), lambda i: (i, 0)),
            pl.BlockSpec((DIM, NUM_EMBEDDINGS), lambda i: (0, 0)),
        ],
        out_specs=[
            pl.BlockSpec((BLOCK_N, DIM), lambda i: (i, 0)),
            pl.BlockSpec((BLOCK_N,), lambda i: (i,)),
            pl.BlockSpec((1, 1), lambda i: (0, 0)),
        ],
        out_shape=[
            jax.ShapeDtypeStruct((n, DIM), jnp.float32),
            jax.ShapeDtypeStruct((n,), jnp.int32),
            jax.ShapeDtypeStruct((1, 1), jnp.float32),
        ],
        scratch_shapes=[pltpu.VMEM((1, NUM_EMBEDDINGS), jnp.float32),
                        pltpu.VMEM((_AUG, NUM_EMBEDDINGS), jnp.float32),
                        pltpu.VMEM((1, NUM_EMBEDDINGS), jnp.float32),
                        pltpu.VMEM((8, NUM_EMBEDDINGS), jnp.float32)],
    )(flat, embed)
    quantize = q.reshape(x.shape)
    diff = acc[0, 0] / jnp.float32(n * DIM)
    embed_ind = ind.reshape(x.shape[:-1])
    return (quantize, diff, embed_ind)


# speedup vs baseline: 1.3826x; 1.3826x over previous
import jax
import jax.numpy as jnp
from jax.experimental import pallas as pl
from jax.experimental.pallas import tpu as pltpu

DIM = 32
NUM_EMBEDDINGS = 8192
TEMP = 4.0
BLOCK_N = 512
N_BLOCKS = 8192 // BLOCK_N
_EXP2_SCALE = 1.4426950408889634 / TEMP
_AUG = 64


def _fused_body(x_ref, xn_ref, embed_ref, q_ref, ind_ref, acc_ref,
                e2_ref, eaug_ref, iota_ref, xm2_ref):
    i = pl.program_id(0)
    xb = x_ref[...]
    emb = embed_ref[...]

    @pl.when(i == 0)
    def _init():
        e2_ref[...] = jnp.sum(emb * emb, axis=0, keepdims=True)
        rows = jax.lax.broadcasted_iota(
            jnp.int32, (_AUG, NUM_EMBEDDINGS), 0)
        eaug_ref[...] = jnp.where(rows == DIM, 1.0, 0.0)
        eaug_ref[0:DIM, :] = emb
        iota_ref[...] = jax.lax.broadcasted_iota(
            jnp.int32, (1, NUM_EMBEDDINGS), 1).astype(jnp.float32)
        acc_ref[...] = jnp.zeros((1, 1), jnp.float32)
        xm2_ref[...] = jax.lax.dot_general(
            xb * (-2.0), emb, (((1,), (0,)), ((), ())),
            preferred_element_type=jnp.float32)

    x2 = jnp.sum(xb * xb, axis=1, keepdims=True)
    dist = (x2 + xm2_ref[...]) + e2_ref[...]

    xm2_ref[...] = jax.lax.dot_general(
        xn_ref[...] * (-2.0), emb, (((1,), (0,)), ((), ())),
        preferred_element_type=jnp.float32)

    m = jnp.min(dist, axis=1, keepdims=True)
    ind_f = jnp.min(
        jnp.where(dist == m, iota_ref[...], float(NUM_EMBEDDINGS)), axis=1)
    ind_ref[...] = ind_f.astype(jnp.int32)

    ez = jnp.exp2(dist * (-_EXP2_SCALE))

    qs = jax.lax.dot_general(
        ez, eaug_ref[...], (((1,), (1,)), ((), ())),
        preferred_element_type=jnp.float32)
    q = qs[:, 0:DIM] / qs[:, DIM:DIM + 1]
    q_ref[...] = q

    acc_ref[...] += jnp.sum((q - xb) ** 2).reshape(1, 1)


def kernel(x, embed):
    n = x.shape[0] * x.shape[1]
    flat = x.reshape(n, DIM)
    grid = (n // BLOCK_N,)
    last = n // BLOCK_N - 1
    q, ind, acc = pl.pallas_call(
        _fused_body,
        grid=grid,
        in_specs=[
            pl.BlockSpec((BLOCK_N, DIM), lambda i: (i, 0)),
            pl.BlockSpec((BLOCK_N, DIM),
                         lambda i: (jnp.minimum(i + 1, last), 0)),
            pl.BlockSpec((DIM, NUM_EMBEDDINGS), lambda i: (0, 0)),
        ],
        out_specs=[
            pl.BlockSpec((BLOCK_N, DIM), lambda i: (i, 0)),
            pl.BlockSpec((BLOCK_N,), lambda i: (i,)),
            pl.BlockSpec((1, 1), lambda i: (0, 0)),
        ],
        out_shape=[
            jax.ShapeDtypeStruct((n, DIM), jnp.float32),
            jax.ShapeDtypeStruct((n,), jnp.int32),
            jax.ShapeDtypeStruct((1, 1), jnp.float32),
        ],
        scratch_shapes=[pltpu.VMEM((1, NUM_EMBEDDINGS), jnp.float32),
                        pltpu.VMEM((_AUG, NUM_EMBEDDINGS), jnp.float32),
                        pltpu.VMEM((1, NUM_EMBEDDINGS), jnp.float32),
                        pltpu.VMEM((BLOCK_N, NUM_EMBEDDINGS), jnp.float32)],
    )(flat, flat, embed)
    quantize = q.reshape(x.shape)
    diff = acc[0, 0] / jnp.float32(n * DIM)
    embed_ind = ind.reshape(x.shape[:-1])
    return (quantize, diff, embed_ind)


# speedup vs baseline: 1.7053x; 1.2334x over previous
import jax
import jax.numpy as jnp
from jax.experimental import pallas as pl
from jax.experimental.pallas import tpu as pltpu

DIM = 32
NUM_EMBEDDINGS = 8192
TEMP = 4.0
BLOCK_N = 1024
_EXP2_SCALE = 1.4426950408889634 / TEMP
_AUG = 64


def _fused_body(x_ref, embed_ref, q_ref, ind_ref, acc_ref, e2_ref, eaug_ref,
                iota_ref):
    i = pl.program_id(0)
    xb = x_ref[...]
    emb = embed_ref[...]

    @pl.when(i == 0)
    def _init():
        e2_ref[...] = jnp.sum(emb * emb, axis=0, keepdims=True)
        rows = jax.lax.broadcasted_iota(
            jnp.int32, (_AUG, NUM_EMBEDDINGS), 0)
        eaug_ref[...] = jnp.where(rows == DIM, 1.0, 0.0)
        eaug_ref[0:DIM, :] = emb
        iota_ref[...] = jax.lax.broadcasted_iota(
            jnp.int32, (1, NUM_EMBEDDINGS), 1).astype(jnp.float32)
        acc_ref[...] = jnp.zeros((1, 1), jnp.float32)

    x2 = jnp.sum(xb * xb, axis=1, keepdims=True)
    xm2 = jax.lax.dot_general(
        xb * (-2.0), emb, (((1,), (0,)), ((), ())),
        preferred_element_type=jnp.float32)
    dist = (x2 + xm2) + e2_ref[...]

    ez = jnp.exp2(dist * (-_EXP2_SCALE))

    qs = jax.lax.dot_general(
        ez, eaug_ref[...], (((1,), (1,)), ((), ())),
        preferred_element_type=jnp.float32)

    m = jnp.min(dist, axis=1, keepdims=True)
    ind_f = jnp.min(
        jnp.where(dist == m, iota_ref[...], float(NUM_EMBEDDINGS)), axis=1)
    ind_ref[...] = ind_f.astype(jnp.int32)

    q = qs[:, 0:DIM] / qs[:, DIM:DIM + 1]
    q_ref[...] = q

    acc_ref[...] += jnp.sum((q - xb) ** 2).reshape(1, 1)


def kernel(x, embed):
    n = x.shape[0] * x.shape[1]
    flat = x.reshape(n, DIM)
    grid = (n // BLOCK_N,)
    q, ind, acc = pl.pallas_call(
        _fused_body,
        grid=grid,
        in_specs=[
            pl.BlockSpec((BLOCK_N, DIM), lambda i: (i, 0)),
            pl.BlockSpec((DIM, NUM_EMBEDDINGS), lambda i: (0, 0)),
        ],
        out_specs=[
            pl.BlockSpec((BLOCK_N, DIM), lambda i: (i, 0)),
            pl.BlockSpec((BLOCK_N,), lambda i: (i,)),
            pl.BlockSpec((1, 1), lambda i: (0, 0)),
        ],
        out_shape=[
            jax.ShapeDtypeStruct((n, DIM), jnp.float32),
            jax.ShapeDtypeStruct((n,), jnp.int32),
            jax.ShapeDtypeStruct((1, 1), jnp.float32),
        ],
        scratch_shapes=[pltpu.VMEM((1, NUM_EMBEDDINGS), jnp.float32),
                        pltpu.VMEM((_AUG, NUM_EMBEDDINGS), jnp.float32),
                        pltpu.VMEM((1, NUM_EMBEDDINGS), jnp.float32)],
    )(flat, embed)
    quantize = q.reshape(x.shape)
    diff = acc[0, 0] / jnp.float32(n * DIM)
    embed_ind = ind.reshape(x.shape[:-1])
    return (quantize, diff, embed_ind)
